# Initial kernel scaffold; baseline (speedup 1.0000x reference)
import jax
import jax.numpy as jnp
from jax.experimental import pallas as pl


def kernel(x, edge_index, edge_attr, batch, emb, W_rbf, rbf_to_hidden, W_embed, W_msg, W_down, W_rbfb, W_bint, W_up, W_node, W_oe, W_o, gamma, beta, p1w, p1b, p2w, p2b):
    raise NotImplementedError("write your pallas kernel here")


# speedup vs baseline: 1.4050x; 1.4050x over previous
import functools

import jax
import jax.numpy as jnp
from jax import lax
from jax.experimental import pallas as pl
from jax.experimental.pallas import tpu as pltpu
from jax.experimental.pallas import tpu_sc as plsc

N = 50000
E = 800000
H = 64
OUT = 64
NBLK = 4
OE = 128
G = 64

NC = 2
NS = 16
LANE = 128
ROWS_PER_SC = 25
CHUNK = ROWS_PER_SC * LANE
NSC = E // CHUNK
RS = 5
CHUNK_S = RS * LANE
NSC_S = E // CHUNK_S
NPT = 3128
N_PAD = NPT * NS

TE = 2000
GE = E // TE
TN = 1000
GN = N // TN

_SC_MESH = dict(core_axis_name="c", subcore_axis_name="s")


def _f32(shape):
    return jax.ShapeDtypeStruct(shape, jnp.float32)


def _sc_chunk_bounds(s, nsc=NSC):
    return (nsc - s - 1) // NS + 1


def _sc_gather(tabL, tabR, idx1):
    @functools.partial(
        pl.kernel,
        mesh=plsc.VectorSubcoreMesh(**_SC_MESH),
        compiler_params=pltpu.CompilerParams(use_tc_tiling_on_sc=False),
        out_type=(_f32((E, 32)), _f32((E, 32))),
        scratch_types=[
            pltpu.VMEM((CHUNK,), jnp.int32),
            pltpu.VMEM((CHUNK, 32), jnp.float32),
            pltpu.SemaphoreType.DMA,
        ],
    )
    def k(tL, tR, idx_hbm, outL, outR, idx_v, rows_v, sem):
        c = lax.axis_index("c")
        s = lax.axis_index("s")

        def run(tab, out):
            def chunk_body(t, carry):
                ebase = (s + t * NS) * CHUNK
                pltpu.sync_copy(idx_hbm.at[pl.ds(ebase, CHUNK)], idx_v)

                def fire(j, carry2):
                    pltpu.async_copy(
                        tab.at[idx_v.at[pl.ds(j * LANE, LANE)]],
                        rows_v.at[pl.ds(j * LANE, LANE)],
                        sem,
                    )
                    return carry2

                lax.fori_loop(0, ROWS_PER_SC, fire, 0)

                def drain(j, carry2):
                    pltpu.make_async_copy(
                        tab.at[idx_v.at[pl.ds(j * LANE, LANE)]],
                        rows_v.at[pl.ds(j * LANE, LANE)],
                        sem,
                    ).wait()
                    return carry2

                lax.fori_loop(0, ROWS_PER_SC, drain, 0)
                pltpu.sync_copy(rows_v, out.at[pl.ds(ebase, CHUNK)])
                return carry

            lax.fori_loop(0, _sc_chunk_bounds(s), chunk_body, 0)

        @pl.when(c == 0)
        def _():
            run(tL, outL)

        @pl.when(c == 1)
        def _():
            run(tR, outR)

    return k(tabL, tabR, idx1)


def _sc_gather_add(t1L, t1R, t2L, t2R, idxa1, idxb1):
    @functools.partial(
        pl.kernel,
        mesh=plsc.VectorSubcoreMesh(**_SC_MESH),
        compiler_params=pltpu.CompilerParams(use_tc_tiling_on_sc=False),
        out_type=(_f32((E, 32)), _f32((E, 32))),
        scratch_types=[
            pltpu.VMEM((CHUNK,), jnp.int32),
            pltpu.VMEM((CHUNK,), jnp.int32),
            pltpu.VMEM((CHUNK, 32), jnp.float32),
            pltpu.SemaphoreType.DMA,
        ],
    )
    def k(a1L, a1R, a2L, a2R, ia_hbm, ib_hbm, outL, outR, ia_v, ib_v, rows_v, sem):
        c = lax.axis_index("c")
        s = lax.axis_index("s")

        def run(tab1, tab2, out):
            def chunk_body(t, carry):
                ebase = (s + t * NS) * CHUNK
                pltpu.sync_copy(ia_hbm.at[pl.ds(ebase, CHUNK)], ia_v)
                pltpu.sync_copy(ib_hbm.at[pl.ds(ebase, CHUNK)], ib_v)

                def fire1(j, carry2):
                    pltpu.async_copy(
                        tab1.at[ia_v.at[pl.ds(j * LANE, LANE)]],
                        rows_v.at[pl.ds(j * LANE, LANE)], sem)
                    return carry2

                lax.fori_loop(0, ROWS_PER_SC, fire1, 0)

                def drain1(j, carry2):
                    pltpu.make_async_copy(
                        tab1.at[ia_v.at[pl.ds(j * LANE, LANE)]],
                        rows_v.at[pl.ds(j * LANE, LANE)], sem).wait()
                    return carry2

                lax.fori_loop(0, ROWS_PER_SC, drain1, 0)

                def fire2(j, carry2):
                    pltpu.async_copy(
                        tab2.at[ib_v.at[pl.ds(j * LANE, LANE)]],
                        rows_v.at[pl.ds(j * LANE, LANE)], sem, add=True)
                    return carry2

                lax.fori_loop(0, ROWS_PER_SC, fire2, 0)

                def drain2(j, carry2):
                    pltpu.make_async_copy(
                        tab2.at[ib_v.at[pl.ds(j * LANE, LANE)]],
                        rows_v.at[pl.ds(j * LANE, LANE)], sem).wait()
                    return carry2

                lax.fori_loop(0, ROWS_PER_SC, drain2, 0)
                pltpu.sync_copy(rows_v, out.at[pl.ds(ebase, CHUNK)])
                return carry

            lax.fori_loop(0, _sc_chunk_bounds(s), chunk_body, 0)

        @pl.when(c == 0)
        def _():
            run(a1L, a2L, outL)

        @pl.when(c == 1)
        def _():
            run(a1R, a2R, outR)

    return k(t1L, t1R, t2L, t2R, idxa1, idxb1)


def _sc_segsum(vL, vR, idx1):
    @functools.partial(
        pl.kernel,
        mesh=plsc.VectorSubcoreMesh(**_SC_MESH),
        compiler_params=pltpu.CompilerParams(use_tc_tiling_on_sc=False),
        out_type=(_f32((N_PAD, 32)), _f32((N_PAD, 32))),
        scratch_types=[
            pltpu.VMEM_SHARED((N_PAD, 32), jnp.float32),
            pltpu.VMEM((CHUNK_S,), jnp.int32),
            pltpu.VMEM((RS, LANE), jnp.int32),
            pltpu.VMEM((CHUNK_S, 32), jnp.float32),
            pltpu.SemaphoreType.DMA,
        ],
    )
    def k(mL, mR, idx_hbm, outL, outR, shared, idxf_v, idx_v, rows_v, sem):
        c = lax.axis_index("c")
        s = lax.axis_index("s")

        z16 = jnp.zeros((16,), jnp.float32)

        def zrow(i, carry):
            rows_v[i, pl.ds(0, 16)] = z16
            rows_v[i, pl.ds(16, 16)] = z16
            return carry

        lax.fori_loop(0, CHUNK_S, zrow, 0)

        def zcopy(t, carry):
            pltpu.sync_copy(rows_v,
                            shared.at[pl.ds(s * NPT + t * CHUNK_S, CHUNK_S)])
            return carry

        lax.fori_loop(0, NPT // CHUNK_S, zcopy, 0)
        pltpu.sync_copy(
            rows_v.at[pl.ds(0, NPT % CHUNK_S)],
            shared.at[pl.ds(s * NPT + (NPT // CHUNK_S) * CHUNK_S,
                            NPT % CHUNK_S)])
        plsc.subcore_barrier()

        def run(src):
            def chunk_body(t, carry):
                ebase = (s + t * NS) * CHUNK_S
                pltpu.sync_copy(idx_hbm.at[pl.ds(ebase, CHUNK_S)], idxf_v)
                pltpu.sync_copy(src.at[pl.ds(ebase, CHUNK_S)], rows_v)

                def restage(r, carry2):
                    v = idxf_v[pl.ds(r * 16, 16)]
                    idx_v[r // 8, pl.ds((r % 8) * 16, 16)] = v
                    return carry2

                lax.fori_loop(0, CHUNK_S // 16, restage, 0)

                def fire(j, carry2):
                    pltpu.async_copy(
                        rows_v.at[pl.ds(j * LANE, LANE)],
                        shared.at[idx_v.at[j]],
                        sem, add=True)
                    return carry2

                lax.fori_loop(0, RS, fire, 0)

                def drain(j, carry2):
                    pltpu.make_async_copy(
                        rows_v.at[pl.ds(j * LANE, LANE)],
                        shared.at[idx_v.at[j]],
                        sem).wait()
                    return carry2

                lax.fori_loop(0, RS, drain, 0)
                return carry

            lax.fori_loop(0, _sc_chunk_bounds(s, NSC_S), chunk_body, 0)

        @pl.when(c == 0)
        def _():
            run(mL)

        @pl.when(c == 1)
        def _():
            run(mR)

        plsc.subcore_barrier()

        def dump(out):
            pltpu.sync_copy(shared.at[pl.ds(s * NPT, NPT)],
                            out.at[pl.ds(s * NPT, NPT)])

        @pl.when(c == 0)
        def _():
            dump(outL)

        @pl.when(c == 1)
        def _():
            dump(outR)

    return k(vL, vR, idx1)


def _silu(v):
    return v * jax.nn.sigmoid(v)


def _dot(a, b):
    return jax.lax.dot_general(a, b, (((1,), (0,)), ((), ())),
                               preferred_element_type=jnp.float32)


def _tc_node_init(x3, emb_pad, We1, We2):
    def body(x_ref, emb_ref, w1_ref, w2_ref, o1L, o1R, o2L, o2R):
        xi = x_ref[0, 0, :].reshape(TN, 1)
        oh = (lax.broadcasted_iota(jnp.int32, (TN, 128), 1) == xi)
        h = _dot(oh.astype(jnp.float32), emb_ref[...])
        hw1 = _dot(h, w1_ref[...])
        hw2 = _dot(h, w2_ref[...])
        o1L[...] = hw1[:, :32]
        o1R[...] = hw1[:, 32:]
        o2L[...] = hw2[:, :32]
        o2R[...] = hw2[:, 32:]

    half = pl.BlockSpec((TN, 32), lambda i: (i, 0))
    return pl.pallas_call(
        body,
        grid=(GN,),
        in_specs=[
            pl.BlockSpec((1, 1, TN), lambda i: (i, 0, 0)),
            pl.BlockSpec((128, H), lambda i: (0, 0)),
            pl.BlockSpec((H, H), lambda i: (0, 0)),
            pl.BlockSpec((H, H), lambda i: (0, 0)),
        ],
        out_specs=[half, half, half, half],
        out_shape=[_f32((N, 32))] * 4,
    )(x3, emb_pad, We1, We2)


def _tc_edge_init(ea8, g1L, g1R, Wrbf8, K3):
    def body(ea_ref, gL_ref, gR_ref, wr_ref, k3_ref, m_ref, rbf_ref):
        rbf = _silu(_dot(ea_ref[...], wr_ref[...]))
        g = jnp.concatenate([gL_ref[...], gR_ref[...]], axis=1)
        m_ref[...] = _silu(g + _dot(rbf, k3_ref[...]))
        rbf_ref[...] = rbf

    return pl.pallas_call(
        body,
        grid=(GE,),
        in_specs=[
            pl.BlockSpec((TE, 8), lambda i: (i, 0)),
            pl.BlockSpec((TE, 32), lambda i: (i, 0)),
            pl.BlockSpec((TE, 32), lambda i: (i, 0)),
            pl.BlockSpec((8, 8), lambda i: (0, 0)),
            pl.BlockSpec((8, H), lambda i: (0, 0)),
        ],
        out_specs=[
            pl.BlockSpec((TE, H), lambda i: (i, 0)),
            pl.BlockSpec((TE, 8), lambda i: (i, 0)),
        ],
        out_shape=[_f32((E, H)), _f32((E, 8))],
    )(ea8, g1L, g1R, Wrbf8, K3)


def _tc_msg_chain(m, rbf8, Wmsg, Wdown, C8, Wup):
    def body(m_ref, rbf_ref, wm_ref, wd_ref, c8_ref, wu_ref, oL, oR):
        m0 = m_ref[...]
        mm = _silu(_dot(m0, wm_ref[...]))
        md = _silu(_dot(mm, wd_ref[...])) * _dot(rbf_ref[...], c8_ref[...])
        m1 = m0 + _silu(_dot(md, wu_ref[...]))
        oL[...] = m1[:, :32]
        oR[...] = m1[:, 32:]

    half = pl.BlockSpec((TE, 32), lambda i: (i, 0))
    return pl.pallas_call(
        body,
        grid=(GE,),
        in_specs=[
            pl.BlockSpec((TE, H), lambda i: (i, 0)),
            pl.BlockSpec((TE, 8), lambda i: (i, 0)),
            pl.BlockSpec((H, H), lambda i: (0, 0)),
            pl.BlockSpec((H, H), lambda i: (0, 0)),
            pl.BlockSpec((8, H), lambda i: (0, 0)),
            pl.BlockSpec((H, H), lambda i: (0, 0)),
        ],
        out_specs=[half, half],
        out_shape=[_f32((E, 32))] * 2,
    )(m, rbf8, Wmsg, Wdown, C8, Wup)


def _tc_out_chain(m1L, m1R, gL, gR, Wnode, Woe, Wo, yacc):
    with_acc = yacc is not None

    def body(*refs):
        if with_acc:
            (mL_ref, mR_ref, gL_ref, gR_ref, yaL_ref, yaR_ref,
             wn_ref, we_ref, wo_ref, m2_ref, yL, yR) = refs
        else:
            (mL_ref, mR_ref, gL_ref, gR_ref,
             wn_ref, we_ref, wo_ref, m2_ref, yL, yR) = refs
        m1 = jnp.concatenate([mL_ref[...], mR_ref[...]], axis=1)
        g = jnp.concatenate([gL_ref[...], gR_ref[...]], axis=1)
        m2 = m1 + _silu(_dot(g, wn_ref[...]))
        oe = _silu(_dot(m2, we_ref[...]))
        y = _dot(oe, wo_ref[...])
        m2_ref[...] = m2
        if with_acc:
            yL[...] = yaL_ref[...] + y[:, :32]
            yR[...] = yaR_ref[...] + y[:, 32:]
        else:
            yL[...] = y[:, :32]
            yR[...] = y[:, 32:]

    half = pl.BlockSpec((TE, 32), lambda i: (i, 0))
    acc_specs = [half, half] if with_acc else []
    acc_args = list(yacc) if with_acc else []
    return pl.pallas_call(
        body,
        grid=(GE,),
        in_specs=[half, half, half, half] + acc_specs + [
                  pl.BlockSpec((H, H), lambda i: (0, 0)),
                  pl.BlockSpec((H, OE), lambda i: (0, 0)),
                  pl.BlockSpec((OE, H), lambda i: (0, 0))],
        out_specs=[pl.BlockSpec((TE, H), lambda i: (i, 0)), half, half],
        out_shape=[_f32((E, H)), _f32((E, 32)), _f32((E, 32))],
    )(m1L, m1R, gL, gR, *acc_args, Wnode, Woe, Wo)


def _tc_pool_head(no_halves, batch3, gamma2, beta2, p1w, p1b2, p2w8, p2b8):
    n_no = len(no_halves)

    def body(*refs):
        no_refs = refs[:n_no]
        b_ref, gm_ref, bt_ref, w1_ref, b1_ref, w2_ref, b2_ref = refs[n_no:n_no + 7]
        out_ref, acc_ref, cnt_ref = refs[n_no + 7:]
        i = pl.program_id(0)

        @pl.when(i == 0)
        def _():
            acc_ref[...] = jnp.zeros((G, OUT), jnp.float32)
            cnt_ref[...] = jnp.zeros((G, 8), jnp.float32)

        half_sum_L = no_refs[0][...]
        half_sum_R = no_refs[1][...]
        for t in range(2, n_no, 2):
            half_sum_L = half_sum_L + no_refs[t][...]
            half_sum_R = half_sum_R + no_refs[t + 1][...]
        node_out = jnp.concatenate([half_sum_L, half_sum_R], axis=1)

        bi = b_ref[0, 0, :].reshape(TN, 1)
        ohb = (lax.broadcasted_iota(jnp.int32, (TN, G), 1) == bi).astype(jnp.float32)
        acc_ref[...] += jax.lax.dot_general(
            ohb, node_out, (((0,), (0,)), ((), ())),
            preferred_element_type=jnp.float32)
        cnt_ref[...] += jax.lax.dot_general(
            ohb, jnp.ones((TN, 8), jnp.float32), (((0,), (0,)), ((), ())),
            preferred_element_type=jnp.float32)

        @pl.when(i == GN - 1)
        def _():
            cnt = jnp.maximum(cnt_ref[...][:, 0:1], 1.0)
            pooled = acc_ref[...] / cnt
            mean = jnp.mean(pooled, axis=1, keepdims=True)
            ctr = pooled - mean
            var = jnp.mean(ctr * ctr, axis=1, keepdims=True)
            z = ctr * jax.lax.rsqrt(var + 1e-5) * gm_ref[...] + bt_ref[...]
            hdn = jnp.maximum(_dot(z, w1_ref[...]) + b1_ref[...], 0.0)
            out_ref[...] = _dot(hdn, w2_ref[...]) + b2_ref[...]

    half = pl.BlockSpec((TN, 32), lambda i: (i, 0))
    return pl.pallas_call(
        body,
        grid=(GN,),
        in_specs=[half] * n_no + [
            pl.BlockSpec((1, 1, TN), lambda i: (i, 0, 0)),
            pl.BlockSpec((1, OUT), lambda i: (0, 0)),
            pl.BlockSpec((1, OUT), lambda i: (0, 0)),
            pl.BlockSpec((OUT, 128), lambda i: (0, 0)),
            pl.BlockSpec((1, 128), lambda i: (0, 0)),
            pl.BlockSpec((128, 8), lambda i: (0, 0)),
            pl.BlockSpec((1, 8), lambda i: (0, 0)),
        ],
        out_specs=pl.BlockSpec((G, 8), lambda i: (0, 0)),
        out_shape=_f32((G, 8)),
        scratch_shapes=[pltpu.VMEM((G, OUT), jnp.float32),
                        pltpu.VMEM((G, 8), jnp.float32)],
    )(*no_halves, batch3, gamma2, beta2, p1w, p1b2, p2w8, p2b8)


def kernel(x, edge_index, edge_attr, batch, emb, W_rbf, rbf_to_hidden, W_embed,
           W_msg, W_down, W_rbfb, W_bint, W_up, W_node, W_oe, W_o,
           gamma, beta, p1w, p1b, p2w, p2b):
    f32 = jnp.float32
    x3 = x.astype(jnp.int32).reshape(GN, 1, TN)
    batch3 = batch.astype(jnp.int32).reshape(GN, 1, TN)
    src1 = edge_index[0].astype(jnp.int32)
    dst1 = edge_index[1].astype(jnp.int32)

    emb_pad = jnp.zeros((128, H), f32).at[:emb.shape[0]].set(emb)
    We1 = W_embed[:H]
    We2 = W_embed[H:2 * H]
    K3 = jnp.zeros((8, H), f32).at[:rbf_to_hidden.shape[0]].set(
        rbf_to_hidden @ W_embed[2 * H:])
    Wrbf8 = jnp.zeros((8, 8), f32).at[:4, :W_rbf.shape[1]].set(W_rbf)
    C8 = jnp.zeros((NBLK, 8, H), f32).at[:, :W_rbfb.shape[1]].set(
        jnp.einsum("bnk,bkh->bnh", W_rbfb, W_bint))
    ea8 = jnp.zeros((E, 8), f32).at[:, :4].set(edge_attr)

    gamma2 = gamma.reshape(1, OUT)
    beta2 = beta.reshape(1, OUT)
    p1b2 = p1b.reshape(1, 128)
    p2w8 = jnp.zeros((128, 8), f32).at[:, :4].set(p2w)
    p2b8 = jnp.zeros((1, 8), f32).at[0, :4].set(p2b)

    hW1L, hW1R, hW2L, hW2R = _tc_node_init(x3, emb_pad, We1, We2)
    g1L, g1R = _sc_gather_add(hW1L, hW1R, hW2L, hW2R, src1, dst1)
    m, rbf8 = _tc_edge_init(ea8, g1L, g1R, Wrbf8, K3)

    yacc = None
    for b in range(NBLK):
        m1L, m1R = _tc_msg_chain(m, rbf8, W_msg[b], W_down[b], C8[b], W_up[b])
        aggL, aggR = _sc_segsum(m1L, m1R, dst1)
        gL, gR = _sc_gather(aggL, aggR, src1)
        m, yL, yR = _tc_out_chain(m1L, m1R, gL, gR, W_node[b], W_oe[b],
                                  W_o[b], yacc)
        yacc = (yL, yR)

    noL, noR = _sc_segsum(yacc[0], yacc[1], dst1)
    out8 = _tc_pool_head([noL, noR], batch3, gamma2, beta2, p1w, p1b2, p2w8, p2b8)
    return out8[:, :4]
